# Initial kernel scaffold; baseline (speedup 1.0000x reference)
import functools

import jax
import jax.numpy as jnp
from jax import lax
from jax.experimental import pallas as pl
from jax.experimental.pallas import tpu as pltpu
from jax.experimental.pallas import tpu_sc as plsc

NC = 2
NS = 16
NW = NC * NS
EG = 128
LANES = 16


def _sc_edge_pass(npad, gpw, src2d, dst2d, a_src, a_dst, h):
    c = h.shape[1]
    njg = c // LANES
    rows_per_tile = npad // NS
    zlen = 640
    zlast = npad - (NS - 1) * zlen

    mesh = plsc.VectorSubcoreMesh(core_axis_name="c", subcore_axis_name="s")

    @functools.partial(
        pl.kernel,
        out_type=(
            jax.ShapeDtypeStruct((NC, npad, c), jnp.float32),
            jax.ShapeDtypeStruct((NC, npad), jnp.float32),
        ),
        mesh=mesh,
        scratch_types=[
            pltpu.VMEM((gpw, EG), jnp.int32),
            pltpu.VMEM((gpw, EG), jnp.int32),
            pltpu.VMEM((npad,), jnp.float32),
            pltpu.VMEM((npad,), jnp.float32),
            pltpu.VMEM((EG,), jnp.float32),
            pltpu.VMEM((EG, c), jnp.float32),
            pltpu.VMEM((640,), jnp.float32),
            pltpu.VMEM_SHARED((npad, c), jnp.float32),
            pltpu.VMEM_SHARED((npad,), jnp.float32),
        ],
    )
    def k(src_hbm, dst_hbm, as_hbm, ad_hbm, h_hbm, num_out, den_out,
          src_t, dst_t, as_t, ad_t, wbuf, rowbuf, zrow, acc_num, acc_den):
        cid = lax.axis_index("c")
        sid = lax.axis_index("s")
        wid = sid * NC + cid

        pltpu.sync_copy(src_hbm.at[pl.ds(wid * gpw, gpw)], src_t)
        pltpu.sync_copy(dst_hbm.at[pl.ds(wid * gpw, gpw)], dst_t)
        pltpu.sync_copy(as_hbm, as_t)
        pltpu.sync_copy(ad_hbm, ad_t)

        zv = jnp.zeros((LANES,), jnp.float32)

        def zero_zrow(i, carry):
            zrow[pl.ds(i * LANES, LANES)] = zv
            return carry

        lax.fori_loop(0, 640 // LANES, zero_zrow, 0)

        def zero_rowbuf(i, carry):
            for j in range(njg):
                rowbuf[i, pl.ds(j * LANES, LANES)] = zv
            return carry

        lax.fori_loop(0, EG, zero_rowbuf, 0)

        nbase = sid * rows_per_tile
        nfull = rows_per_tile // EG
        for t in range(nfull):
            pltpu.sync_copy(rowbuf, acc_num.at[pl.ds(nbase + t * EG, EG)])
        rem = rows_per_tile - nfull * EG
        if rem:
            pltpu.sync_copy(rowbuf.at[pl.ds(0, rem)],
                            acc_num.at[pl.ds(nbase + nfull * EG, rem)])

        @pl.when(sid < NS - 1)
        def _():
            pltpu.sync_copy(zrow, acc_den.at[pl.ds(sid * zlen, zlen)])

        @pl.when(sid == NS - 1)
        def _():
            pltpu.sync_copy(zrow.at[pl.ds(0, zlast)],
                            acc_den.at[pl.ds((NS - 1) * zlen, zlast)])

        plsc.subcore_barrier()

        def group_body(g, carry):
            for i in range(EG // LANES):
                srcv = src_t[g, pl.ds(i * LANES, LANES)]
                dstv = dst_t[g, pl.ds(i * LANES, LANES)]
                a = (plsc.load_gather(as_t, [srcv])
                     + plsc.load_gather(ad_t, [dstv]))
                a = jnp.where(a >= 0.0, a, 0.2 * a)
                wbuf[pl.ds(i * LANES, LANES)] = jnp.exp(a)

            pltpu.sync_copy(h_hbm.at[src_t.at[g]], rowbuf)

            def scale_body(e, carry2):
                ws = wbuf[e]
                for j in range(njg):
                    rowbuf[e, pl.ds(j * LANES, LANES)] = (
                        rowbuf[e, pl.ds(j * LANES, LANES)] * ws)
                return carry2

            lax.fori_loop(0, EG, scale_body, 0)

            pltpu.sync_copy(rowbuf, acc_num.at[dst_t.at[g]], add=True)
            pltpu.sync_copy(wbuf, acc_den.at[dst_t.at[g]], add=True)
            return carry

        lax.fori_loop(0, gpw, group_body, 0)

        plsc.subcore_barrier()

        pltpu.sync_copy(acc_num.at[pl.ds(nbase, rows_per_tile)],
                        num_out.at[cid].at[pl.ds(nbase, rows_per_tile)])

        @pl.when(sid < NS - 1)
        def _():
            pltpu.sync_copy(acc_den.at[pl.ds(sid * zlen, zlen)],
                            den_out.at[cid].at[pl.ds(sid * zlen, zlen)])

        @pl.when(sid == NS - 1)
        def _():
            pltpu.sync_copy(acc_den.at[pl.ds((NS - 1) * zlen, zlast)],
                            den_out.at[cid].at[pl.ds((NS - 1) * zlen, zlast)])

    return k(src2d, dst2d, a_src, a_dst, h)


def _tc_prep(xp, w, att_s, att_d):
    npad = xp.shape[0]
    c = w.shape[1]

    def body(x_ref, w_ref, s_ref, d_ref, h_ref, as_ref, ad_ref):
        h = jnp.dot(x_ref[...], w_ref[...], preferred_element_type=jnp.float32)
        h_ref[...] = h
        as_ref[...] = jnp.sum(h * s_ref[...], axis=1, keepdims=True)
        ad_ref[...] = jnp.sum(h * d_ref[...], axis=1, keepdims=True)

    return pl.pallas_call(
        body,
        out_shape=(
            jax.ShapeDtypeStruct((npad, c), jnp.float32),
            jax.ShapeDtypeStruct((npad, 1), jnp.float32),
            jax.ShapeDtypeStruct((npad, 1), jnp.float32),
        ),
    )(xp, w, att_s, att_d)


def _combine(num_ref, den_ref, h_ref, as_ref, ad_ref, b_ref):
    num = num_ref[0] + num_ref[1]
    den = den_ref[0] + den_ref[1]
    a = as_ref[...] + ad_ref[...]
    wself = jnp.exp(jnp.where(a >= 0.0, a, 0.2 * a))
    num = num + wself * h_ref[...]
    den = den + wself
    return jnp.maximum(num / (den + 1e-16) + b_ref[...], 0.0)


def _tc_mid(n_real, num, den, h, a_s, a_d, b, gamma, beta, w1, att_s1, att_d1):
    npad, c = h.shape

    def body(num_ref, den_ref, h_ref, as_ref, ad_ref, b_ref, g_ref, be_ref,
             w1_ref, s1_ref, d1_ref, h1_ref, as1_ref, ad1_ref):
        z = _combine(num_ref, den_ref, h_ref, as_ref, ad_ref, b_ref)
        zz = z[0:n_real, :]
        mean = jnp.mean(zz, axis=0, keepdims=True)
        var = jnp.mean((zz - mean) * (zz - mean), axis=0, keepdims=True)
        zn = (zz - mean) * lax.rsqrt(var + 1e-5) * g_ref[...] + be_ref[...]
        h1 = jnp.dot(zn, w1_ref[...], preferred_element_type=jnp.float32)
        h1_ref[0:n_real, :] = h1
        h1_ref[n_real:npad, :] = jnp.zeros((npad - n_real, c), jnp.float32)
        as1 = jnp.sum(h1 * s1_ref[...], axis=1, keepdims=True)
        ad1 = jnp.sum(h1 * d1_ref[...], axis=1, keepdims=True)
        zpad = jnp.zeros((npad - n_real, 1), jnp.float32)
        as1_ref[0:n_real, :] = as1
        as1_ref[n_real:npad, :] = zpad
        ad1_ref[0:n_real, :] = ad1
        ad1_ref[n_real:npad, :] = zpad

    return pl.pallas_call(
        body,
        out_shape=(
            jax.ShapeDtypeStruct((npad, c), jnp.float32),
            jax.ShapeDtypeStruct((npad, 1), jnp.float32),
            jax.ShapeDtypeStruct((npad, 1), jnp.float32),
        ),
    )(num, den, h, a_s, a_d, b, gamma, beta, w1, att_s1, att_d1)


def _tc_final(n_real, num, den, h, a_s, a_d, b):
    c = h.shape[1]

    def body(num_ref, den_ref, h_ref, as_ref, ad_ref, b_ref, out_ref):
        z = _combine(num_ref, den_ref, h_ref, as_ref, ad_ref, b_ref)
        out_ref[...] = z[0:n_real, :]

    return pl.pallas_call(
        body,
        out_shape=jax.ShapeDtypeStruct((n_real, c), jnp.float32),
    )(num, den, h, a_s, a_d, b)


def kernel(x, edge_index, W0, att_src0, att_dst0, b0, gamma0, beta0,
           W1, att_src1, att_dst1, b1):
    n, d = x.shape
    c = W0.shape[1]
    e = edge_index.shape[1]

    trash = ((n + 7) // 8) * 8
    npad = ((trash + 16 + NS - 1) // NS) * NS
    groups = (e + EG - 1) // EG
    gpw = (groups + NW - 1) // NW
    epad = NW * gpw * EG

    x = x.astype(jnp.float32)
    ei = edge_index.astype(jnp.int32)
    src = jnp.concatenate([ei[0], jnp.zeros((epad - e,), jnp.int32)])
    dst = jnp.concatenate([ei[1], jnp.full((epad - e,), trash, jnp.int32)])
    src2d = src.reshape(NW * gpw, EG)
    dst2d = dst.reshape(NW * gpw, EG)
    xp = jnp.concatenate([x, jnp.zeros((npad - n, d), jnp.float32)], axis=0)

    as_r0 = att_src0.reshape(1, c)
    ad_r0 = att_dst0.reshape(1, c)
    as_r1 = att_src1.reshape(1, c)
    ad_r1 = att_dst1.reshape(1, c)
    b0r = b0.reshape(1, c)
    b1r = b1.reshape(1, c)
    g0r = gamma0.reshape(1, c)
    be0r = beta0.reshape(1, c)

    h0, as0, ad0 = _tc_prep(xp, W0, as_r0, ad_r0)
    num0, den0 = _sc_edge_pass(npad, gpw, src2d, dst2d,
                               as0.reshape(npad), ad0.reshape(npad), h0)
    h1, as1, ad1 = _tc_mid(n, num0, den0.reshape(NC, npad, 1), h0, as0, ad0,
                           b0r, g0r, be0r, W1, as_r1, ad_r1)
    num1, den1 = _sc_edge_pass(npad, gpw, src2d, dst2d,
                               as1.reshape(npad), ad1.reshape(npad), h1)
    return _tc_final(n, num1, den1.reshape(NC, npad, 1), h1, as1, ad1, b1r)


# speedup vs baseline: 14.8423x; 14.8423x over previous
import functools

import jax
import jax.numpy as jnp
from jax import lax
from jax.experimental import pallas as pl
from jax.experimental.pallas import tpu as pltpu
from jax.experimental.pallas import tpu_sc as plsc

NC = 2
NS = 16
NW = NC * NS
EG = 128
WIN = 8
LANES = 16


def _sc_edge_pass(npad, gpw, src2d, dst2d, a_src, a_dst, h):
    c = h.shape[1]
    njg = c // LANES
    rows_per_tile = npad // NS
    blocks = npad // 128
    bpt = blocks // NS
    zrem = blocks % NS
    zbuf = (bpt + 1) * 128 if zrem else bpt * 128

    mesh = plsc.VectorSubcoreMesh(core_axis_name="c", subcore_axis_name="s",
                                  num_cores=NC, num_subcores=NS)

    @functools.partial(
        pl.kernel,
        out_type=(
            jax.ShapeDtypeStruct((NC, npad, c), jnp.float32),
            jax.ShapeDtypeStruct((NC, npad), jnp.float32),
        ),
        mesh=mesh,
        compiler_params=pltpu.CompilerParams(needs_layout_passes=False),
        scratch_types=[
            pltpu.VMEM((WIN, EG), jnp.int32),
            pltpu.VMEM((WIN, EG), jnp.int32),
            pltpu.VMEM((npad,), jnp.float32),
            pltpu.VMEM((npad,), jnp.float32),
            pltpu.VMEM((EG,), jnp.float32),
            pltpu.VMEM((EG, c), jnp.float32),
            pltpu.VMEM((zbuf,), jnp.float32),
            pltpu.VMEM_SHARED((npad, c), jnp.float32),
            pltpu.VMEM_SHARED((npad,), jnp.float32),
        ],
    )
    def k(src_hbm, dst_hbm, as_hbm, ad_hbm, h_hbm, num_out, den_out,
          src_t, dst_t, as_t, ad_t, wbuf, rowbuf, zrow, acc_num, acc_den):
        cid = lax.axis_index("c")
        sid = lax.axis_index("s")
        wid = sid * NC + cid

        pltpu.sync_copy(as_hbm, as_t)
        pltpu.sync_copy(ad_hbm, ad_t)

        zv = jnp.zeros((LANES,), jnp.float32)

        def zero_zrow(i, carry):
            zrow[pl.ds(i * LANES, LANES)] = zv
            return carry

        lax.fori_loop(0, zbuf // LANES, zero_zrow, 0)

        def zero_rowbuf(i, carry):
            for j in range(njg):
                rowbuf[i, pl.ds(j * LANES, LANES)] = zv
            return carry

        lax.fori_loop(0, EG, zero_rowbuf, 0)

        nbase = sid * rows_per_tile
        nfull = rows_per_tile // EG
        for t in range(nfull):
            pltpu.sync_copy(rowbuf, acc_num.at[pl.ds(nbase + t * EG, EG)])
        rem = rows_per_tile - nfull * EG
        if rem:
            pltpu.sync_copy(rowbuf.at[pl.ds(0, rem)],
                            acc_num.at[pl.ds(nbase + nfull * EG, rem)])

        zoff_hi = sid * (bpt + 1) * 128
        zoff_lo = (zrem * (bpt + 1) + (sid - zrem) * bpt) * 128

        if zrem:
            @pl.when(sid < zrem)
            def _():
                pltpu.sync_copy(zrow.at[pl.ds(0, (bpt + 1) * 128)],
                                acc_den.at[pl.ds(zoff_hi, (bpt + 1) * 128)])

        if bpt:
            @pl.when(sid >= zrem)
            def _():
                pltpu.sync_copy(zrow.at[pl.ds(0, bpt * 128)],
                                acc_den.at[pl.ds(zoff_lo, bpt * 128)])

        plsc.subcore_barrier()

        def super_body(sg, carry):
            pltpu.sync_copy(src_hbm.at[pl.ds(wid * gpw + sg * WIN, WIN)],
                            src_t)
            pltpu.sync_copy(dst_hbm.at[pl.ds(wid * gpw + sg * WIN, WIN)],
                            dst_t)
            lax.fori_loop(0, WIN, group_body, 0)
            return carry

        def group_body(g, carry):
            for i in range(EG // LANES):
                srcv = src_t[g, pl.ds(i * LANES, LANES)]
                dstv = dst_t[g, pl.ds(i * LANES, LANES)]
                a = (plsc.load_gather(as_t, [srcv])
                     + plsc.load_gather(ad_t, [dstv]))
                a = jnp.where(a >= 0.0, a, 0.2 * a)
                wbuf[pl.ds(i * LANES, LANES)] = jnp.exp(a)

            pltpu.sync_copy(h_hbm.at[src_t.at[g]], rowbuf)

            def scale_body(i, carry2):
                wv = wbuf[pl.ds(i * LANES, LANES)]
                for l in range(LANES):
                    eidx = i * LANES + l
                    ws = wv[l]
                    for j in range(njg):
                        rowbuf[eidx, pl.ds(j * LANES, LANES)] = (
                            rowbuf[eidx, pl.ds(j * LANES, LANES)] * ws)
                return carry2

            lax.fori_loop(0, EG // LANES, scale_body, 0)

            pltpu.sync_copy(rowbuf, acc_num.at[dst_t.at[g]], add=True)
            pltpu.sync_copy(wbuf, acc_den.at[dst_t.at[g]], add=True)
            return carry

        lax.fori_loop(0, gpw // WIN, super_body, 0)

        plsc.subcore_barrier()

        pltpu.sync_copy(acc_num.at[pl.ds(nbase, rows_per_tile)],
                        num_out.at[cid].at[pl.ds(nbase, rows_per_tile)])

        if zrem:
            @pl.when(sid < zrem)
            def _():
                pltpu.sync_copy(
                    acc_den.at[pl.ds(zoff_hi, (bpt + 1) * 128)],
                    den_out.at[cid].at[pl.ds(zoff_hi, (bpt + 1) * 128)])

        if bpt:
            @pl.when(sid >= zrem)
            def _():
                pltpu.sync_copy(acc_den.at[pl.ds(zoff_lo, bpt * 128)],
                                den_out.at[cid].at[pl.ds(zoff_lo, bpt * 128)])

    return k(src2d, dst2d, a_src, a_dst, h)


def _tc_prep(xp, w, att_s, att_d):
    npad = xp.shape[0]
    c = w.shape[1]

    def body(x_ref, w_ref, s_ref, d_ref, h_ref, as_ref, ad_ref):
        h = jnp.dot(x_ref[...], w_ref[...], preferred_element_type=jnp.float32)
        h_ref[...] = h
        as_ref[...] = jnp.sum(h * s_ref[...], axis=1, keepdims=True)
        ad_ref[...] = jnp.sum(h * d_ref[...], axis=1, keepdims=True)

    return pl.pallas_call(
        body,
        out_shape=(
            jax.ShapeDtypeStruct((npad, c), jnp.float32),
            jax.ShapeDtypeStruct((npad, 1), jnp.float32),
            jax.ShapeDtypeStruct((npad, 1), jnp.float32),
        ),
    )(xp, w, att_s, att_d)


def _combine(num_ref, den_ref, h_ref, as_ref, ad_ref, b_ref):
    num = num_ref[0] + num_ref[1]
    den = den_ref[0] + den_ref[1]
    a = as_ref[...] + ad_ref[...]
    wself = jnp.exp(jnp.where(a >= 0.0, a, 0.2 * a))
    num = num + wself * h_ref[...]
    den = den + wself
    return jnp.maximum(num / (den + 1e-16) + b_ref[...], 0.0)


def _tc_mid(n_real, num, den, h, a_s, a_d, b, gamma, beta, w1, att_s1, att_d1):
    npad, c = h.shape

    def body_a(num_ref, den_ref, h_ref, as_ref, ad_ref, b_ref,
               z_ref, s1_ref, s2_ref):
        z = _combine(num_ref, den_ref, h_ref, as_ref, ad_ref, b_ref)
        z_ref[...] = z
        zz = z[0:n_real, :]
        s1_ref[...] = jnp.sum(zz, axis=0, keepdims=True)
        s2_ref[...] = jnp.sum(zz * zz, axis=0, keepdims=True)

    z, zsum, zsq = pl.pallas_call(
        body_a,
        out_shape=(
            jax.ShapeDtypeStruct((npad, c), jnp.float32),
            jax.ShapeDtypeStruct((1, c), jnp.float32),
            jax.ShapeDtypeStruct((1, c), jnp.float32),
        ),
    )(num, den, h, a_s, a_d, b)

    def body_b(z_ref, s1_ref, s2_ref, g_ref, be_ref, w1_ref, as1w_ref,
               ad1w_ref, h1_ref, as1_ref, ad1_ref):
        mean = s1_ref[...] * (1.0 / n_real)
        var = s2_ref[...] * (1.0 / n_real) - mean * mean
        zz = z_ref[0:n_real, :]
        zn = (zz - mean) * lax.rsqrt(var + 1e-5) * g_ref[...] + be_ref[...]
        h1 = jnp.dot(zn, w1_ref[...], preferred_element_type=jnp.float32)
        h1_ref[0:n_real, :] = h1
        h1_ref[n_real:npad, :] = jnp.zeros((npad - n_real, c), jnp.float32)
        as1 = jnp.sum(h1 * as1w_ref[...], axis=1, keepdims=True)
        ad1 = jnp.sum(h1 * ad1w_ref[...], axis=1, keepdims=True)
        zpad = jnp.zeros((npad - n_real, 1), jnp.float32)
        as1_ref[0:n_real, :] = as1
        as1_ref[n_real:npad, :] = zpad
        ad1_ref[0:n_real, :] = ad1
        ad1_ref[n_real:npad, :] = zpad

    return pl.pallas_call(
        body_b,
        out_shape=(
            jax.ShapeDtypeStruct((npad, c), jnp.float32),
            jax.ShapeDtypeStruct((npad, 1), jnp.float32),
            jax.ShapeDtypeStruct((npad, 1), jnp.float32),
        ),
    )(z, zsum, zsq, gamma, beta, w1, att_s1, att_d1)


def _tc_final(n_real, num, den, h, a_s, a_d, b):
    c = h.shape[1]

    def body(num_ref, den_ref, h_ref, as_ref, ad_ref, b_ref, out_ref):
        z = _combine(num_ref, den_ref, h_ref, as_ref, ad_ref, b_ref)
        out_ref[...] = z[0:n_real, :]

    return pl.pallas_call(
        body,
        out_shape=jax.ShapeDtypeStruct((n_real, c), jnp.float32),
    )(num, den, h, a_s, a_d, b)


def kernel(x, edge_index, W0, att_src0, att_dst0, b0, gamma0, beta0,
           W1, att_src1, att_dst1, b1):
    n, d = x.shape
    c = W0.shape[1]
    e = edge_index.shape[1]

    trash = ((n + 7) // 8) * 8
    rpt = ((trash + 16 + NS - 1) // NS + 7) // 8 * 8
    npad = rpt * NS
    groups = (e + EG - 1) // EG
    gpw = (groups + NW - 1) // NW
    gpw = ((gpw + 7) // 8) * 8
    epad = NW * gpw * EG

    x = x.astype(jnp.float32)
    ei = edge_index.astype(jnp.int32)
    src = jnp.concatenate([ei[0], jnp.zeros((epad - e,), jnp.int32)])
    dst = jnp.concatenate([ei[1], jnp.full((epad - e,), trash, jnp.int32)])
    src2d = src.reshape(NW * gpw, EG)
    dst2d = dst.reshape(NW * gpw, EG)
    xp = jnp.concatenate([x, jnp.zeros((npad - n, d), jnp.float32)], axis=0)

    as_r0 = att_src0.reshape(1, c)
    ad_r0 = att_dst0.reshape(1, c)
    as_r1 = att_src1.reshape(1, c)
    ad_r1 = att_dst1.reshape(1, c)
    b0r = b0.reshape(1, c)
    b1r = b1.reshape(1, c)
    g0r = gamma0.reshape(1, c)
    be0r = beta0.reshape(1, c)

    h0, as0, ad0 = _tc_prep(xp, W0, as_r0, ad_r0)
    num0, den0 = _sc_edge_pass(npad, gpw, src2d, dst2d,
                               as0.reshape(npad), ad0.reshape(npad), h0)
    h1, as1, ad1 = _tc_mid(n, num0, den0.reshape(NC, npad, 1), h0, as0, ad0,
                           b0r, g0r, be0r, W1, as_r1, ad_r1)
    num1, den1 = _sc_edge_pass(npad, gpw, src2d, dst2d,
                               as1.reshape(npad), ad1.reshape(npad), h1)
    return _tc_final(n, num1, den1.reshape(NC, npad, 1), h1, as1, ad1, b1r)


# speedup vs baseline: 17.1623x; 1.1563x over previous
import functools

import jax
import jax.numpy as jnp
from jax import lax
from jax.experimental import pallas as pl
from jax.experimental.pallas import tpu as pltpu
from jax.experimental.pallas import tpu_sc as plsc

NC = 2
NS = 16
NW = NC * NS
EG = 128
WIN = 8
LANES = 16


def _sc_edge_pass(npad, gpw, src2d, dst2d, a_src, a_dst, h):
    c = h.shape[1]
    njg = c // LANES
    rows_per_tile = npad // NS
    blocks = npad // 128
    bpt = blocks // NS
    zrem = blocks % NS
    zbuf = (bpt + 1) * 128 if zrem else bpt * 128

    mesh = plsc.VectorSubcoreMesh(core_axis_name="c", subcore_axis_name="s",
                                  num_cores=NC, num_subcores=NS)

    @functools.partial(
        pl.kernel,
        out_type=(
            jax.ShapeDtypeStruct((NC, npad, c), jnp.float32),
            jax.ShapeDtypeStruct((NC, npad), jnp.float32),
        ),
        mesh=mesh,
        compiler_params=pltpu.CompilerParams(needs_layout_passes=False),
        scratch_types=[
            pltpu.VMEM((WIN, EG), jnp.int32),
            pltpu.VMEM((WIN, EG), jnp.int32),
            pltpu.VMEM((2, EG), jnp.float32),
            pltpu.VMEM((2, EG), jnp.float32),
            pltpu.VMEM((EG,), jnp.float32),
            pltpu.VMEM((EG,), jnp.float32),
            pltpu.VMEM((EG, c), jnp.float32),
            pltpu.VMEM((EG, c), jnp.float32),
            pltpu.VMEM((zbuf,), jnp.float32),
            pltpu.VMEM_SHARED((npad, c), jnp.float32),
            pltpu.VMEM_SHARED((npad,), jnp.float32),
            pltpu.SemaphoreType.DMA,
            pltpu.SemaphoreType.DMA,
        ],
    )
    def k(src_hbm, dst_hbm, as_hbm, ad_hbm, h_hbm, num_out, den_out,
          src_t, dst_t, asb, adb, wbuf0, wbuf1, rowbuf0, rowbuf1, zrow,
          acc_num, acc_den, sem0, sem1):
        cid = lax.axis_index("c")
        sid = lax.axis_index("s")
        wid = sid * NC + cid
        rowbufs = (rowbuf0, rowbuf1)
        wbufs = (wbuf0, wbuf1)
        sems = (sem0, sem1)

        zv = jnp.zeros((LANES,), jnp.float32)

        def zero_zrow(i, carry):
            zrow[pl.ds(i * LANES, LANES)] = zv
            return carry

        lax.fori_loop(0, zbuf // LANES, zero_zrow, 0)

        def zero_rowbuf(i, carry):
            for j in range(njg):
                rowbuf0[i, pl.ds(j * LANES, LANES)] = zv
            return carry

        lax.fori_loop(0, EG, zero_rowbuf, 0)

        nbase = sid * rows_per_tile
        nfull = rows_per_tile // EG
        for t in range(nfull):
            pltpu.sync_copy(rowbuf0, acc_num.at[pl.ds(nbase + t * EG, EG)])
        rem = rows_per_tile - nfull * EG
        if rem:
            pltpu.sync_copy(rowbuf0.at[pl.ds(0, rem)],
                            acc_num.at[pl.ds(nbase + nfull * EG, rem)])

        zoff_hi = sid * (bpt + 1) * 128
        zoff_lo = (zrem * (bpt + 1) + (sid - zrem) * bpt) * 128

        if zrem:
            @pl.when(sid < zrem)
            def _():
                pltpu.sync_copy(zrow.at[pl.ds(0, (bpt + 1) * 128)],
                                acc_den.at[pl.ds(zoff_hi, (bpt + 1) * 128)])

        if bpt:
            @pl.when(sid >= zrem)
            def _():
                pltpu.sync_copy(zrow.at[pl.ds(0, bpt * 128)],
                                acc_den.at[pl.ds(zoff_lo, bpt * 128)])

        plsc.subcore_barrier()

        def start_gathers(g, slot):
            pltpu.make_async_copy(h_hbm.at[src_t.at[g]],
                                  rowbufs[slot], sems[slot]).start()
            pltpu.make_async_copy(as_hbm.at[src_t.at[g]],
                                  asb.at[slot], sems[slot]).start()
            pltpu.make_async_copy(ad_hbm.at[dst_t.at[g]],
                                  adb.at[slot], sems[slot]).start()

        def wait_gathers(g, slot):
            pltpu.make_async_copy(h_hbm.at[src_t.at[g]],
                                  rowbufs[slot], sems[slot]).wait()
            pltpu.make_async_copy(as_hbm.at[src_t.at[g]],
                                  asb.at[slot], sems[slot]).wait()
            pltpu.make_async_copy(ad_hbm.at[dst_t.at[g]],
                                  adb.at[slot], sems[slot]).wait()

        def process(g, slot):
            rb = rowbufs[slot]
            wb = wbufs[slot]
            for i in range(EG // LANES):
                a = (asb[slot, pl.ds(i * LANES, LANES)]
                     + adb[slot, pl.ds(i * LANES, LANES)])
                a = jnp.where(a >= 0.0, a, 0.2 * a)
                wb[pl.ds(i * LANES, LANES)] = jnp.exp(a)

            def scale_body(i, carry2):
                wv = wb[pl.ds(i * LANES, LANES)]
                for l in range(LANES):
                    eidx = i * LANES + l
                    ws = wv[l]
                    for j in range(njg):
                        rb[eidx, pl.ds(j * LANES, LANES)] = (
                            rb[eidx, pl.ds(j * LANES, LANES)] * ws)
                return carry2

            lax.fori_loop(0, EG // LANES, scale_body, 0)

            pltpu.sync_copy(rb, acc_num.at[dst_t.at[g]], add=True)
            pltpu.sync_copy(wb, acc_den.at[dst_t.at[g]], add=True)

        def super_body(sg, carry):
            base = wid * gpw + sg * WIN
            pltpu.sync_copy(src_hbm.at[pl.ds(base, WIN)], src_t)
            pltpu.sync_copy(dst_hbm.at[pl.ds(base, WIN)], dst_t)
            start_gathers(0, 0)

            def pair_body(i, carry2):
                g = i * 2
                start_gathers(g + 1, 1)
                wait_gathers(g, 0)
                process(g, 0)
                start_gathers(g + 2, 0)
                wait_gathers(g + 1, 1)
                process(g + 1, 1)
                return carry2

            lax.fori_loop(0, WIN // 2 - 1, pair_body, 0)

            gl = WIN - 2
            start_gathers(gl + 1, 1)
            wait_gathers(gl, 0)
            process(gl, 0)
            wait_gathers(gl + 1, 1)
            process(gl + 1, 1)
            return carry

        lax.fori_loop(0, gpw // WIN, super_body, 0)

        plsc.subcore_barrier()

        pltpu.sync_copy(acc_num.at[pl.ds(nbase, rows_per_tile)],
                        num_out.at[cid].at[pl.ds(nbase, rows_per_tile)])

        if zrem:
            @pl.when(sid < zrem)
            def _():
                pltpu.sync_copy(
                    acc_den.at[pl.ds(zoff_hi, (bpt + 1) * 128)],
                    den_out.at[cid].at[pl.ds(zoff_hi, (bpt + 1) * 128)])

        if bpt:
            @pl.when(sid >= zrem)
            def _():
                pltpu.sync_copy(acc_den.at[pl.ds(zoff_lo, bpt * 128)],
                                den_out.at[cid].at[pl.ds(zoff_lo, bpt * 128)])

    return k(src2d, dst2d, a_src, a_dst, h)


def _tc_prep(xp, w, att_s, att_d):
    npad = xp.shape[0]
    c = w.shape[1]

    def body(x_ref, w_ref, s_ref, d_ref, h_ref, as_ref, ad_ref):
        h = jnp.dot(x_ref[...], w_ref[...], preferred_element_type=jnp.float32)
        h_ref[...] = h
        as_ref[...] = jnp.sum(h * s_ref[...], axis=1, keepdims=True)
        ad_ref[...] = jnp.sum(h * d_ref[...], axis=1, keepdims=True)

    return pl.pallas_call(
        body,
        out_shape=(
            jax.ShapeDtypeStruct((npad, c), jnp.float32),
            jax.ShapeDtypeStruct((npad, 1), jnp.float32),
            jax.ShapeDtypeStruct((npad, 1), jnp.float32),
        ),
    )(xp, w, att_s, att_d)


def _combine(num_ref, den_ref, h_ref, as_ref, ad_ref, b_ref):
    num = num_ref[0] + num_ref[1]
    den = den_ref[0] + den_ref[1]
    a = as_ref[...] + ad_ref[...]
    wself = jnp.exp(jnp.where(a >= 0.0, a, 0.2 * a))
    num = num + wself * h_ref[...]
    den = den + wself
    return jnp.maximum(num / (den + 1e-16) + b_ref[...], 0.0)


def _tc_mid(n_real, num, den, h, a_s, a_d, b, gamma, beta, w1, att_s1, att_d1):
    npad, c = h.shape

    def body_a(num_ref, den_ref, h_ref, as_ref, ad_ref, b_ref,
               z_ref, s1_ref, s2_ref):
        z = _combine(num_ref, den_ref, h_ref, as_ref, ad_ref, b_ref)
        z_ref[...] = z
        zz = z[0:n_real, :]
        s1_ref[...] = jnp.sum(zz, axis=0, keepdims=True)
        s2_ref[...] = jnp.sum(zz * zz, axis=0, keepdims=True)

    z, zsum, zsq = pl.pallas_call(
        body_a,
        out_shape=(
            jax.ShapeDtypeStruct((npad, c), jnp.float32),
            jax.ShapeDtypeStruct((1, c), jnp.float32),
            jax.ShapeDtypeStruct((1, c), jnp.float32),
        ),
    )(num, den, h, a_s, a_d, b)

    def body_b(z_ref, s1_ref, s2_ref, g_ref, be_ref, w1_ref, as1w_ref,
               ad1w_ref, h1_ref, as1_ref, ad1_ref):
        mean = s1_ref[...] * (1.0 / n_real)
        var = s2_ref[...] * (1.0 / n_real) - mean * mean
        zz = z_ref[0:n_real, :]
        zn = (zz - mean) * lax.rsqrt(var + 1e-5) * g_ref[...] + be_ref[...]
        h1 = jnp.dot(zn, w1_ref[...], preferred_element_type=jnp.float32)
        h1_ref[0:n_real, :] = h1
        h1_ref[n_real:npad, :] = jnp.zeros((npad - n_real, c), jnp.float32)
        as1 = jnp.sum(h1 * as1w_ref[...], axis=1, keepdims=True)
        ad1 = jnp.sum(h1 * ad1w_ref[...], axis=1, keepdims=True)
        zpad = jnp.zeros((npad - n_real, 1), jnp.float32)
        as1_ref[0:n_real, :] = as1
        as1_ref[n_real:npad, :] = zpad
        ad1_ref[0:n_real, :] = ad1
        ad1_ref[n_real:npad, :] = zpad

    return pl.pallas_call(
        body_b,
        out_shape=(
            jax.ShapeDtypeStruct((npad, c), jnp.float32),
            jax.ShapeDtypeStruct((npad, 1), jnp.float32),
            jax.ShapeDtypeStruct((npad, 1), jnp.float32),
        ),
    )(z, zsum, zsq, gamma, beta, w1, att_s1, att_d1)


def _tc_final(n_real, num, den, h, a_s, a_d, b):
    c = h.shape[1]

    def body(num_ref, den_ref, h_ref, as_ref, ad_ref, b_ref, out_ref):
        z = _combine(num_ref, den_ref, h_ref, as_ref, ad_ref, b_ref)
        out_ref[...] = z[0:n_real, :]

    return pl.pallas_call(
        body,
        out_shape=jax.ShapeDtypeStruct((n_real, c), jnp.float32),
    )(num, den, h, a_s, a_d, b)


def kernel(x, edge_index, W0, att_src0, att_dst0, b0, gamma0, beta0,
           W1, att_src1, att_dst1, b1):
    n, d = x.shape
    c = W0.shape[1]
    e = edge_index.shape[1]

    trash = ((n + 7) // 8) * 8
    rpt = ((trash + 16 + NS - 1) // NS + 7) // 8 * 8
    npad = rpt * NS
    groups = (e + EG - 1) // EG
    gpw = (groups + NW - 1) // NW
    gpw = ((gpw + 7) // 8) * 8
    epad = NW * gpw * EG

    x = x.astype(jnp.float32)
    ei = edge_index.astype(jnp.int32)
    src = jnp.concatenate([ei[0], jnp.zeros((epad - e,), jnp.int32)])
    dst = jnp.concatenate([ei[1], jnp.full((epad - e,), trash, jnp.int32)])
    src2d = src.reshape(NW * gpw, EG)
    dst2d = dst.reshape(NW * gpw, EG)
    xp = jnp.concatenate([x, jnp.zeros((npad - n, d), jnp.float32)], axis=0)

    as_r0 = att_src0.reshape(1, c)
    ad_r0 = att_dst0.reshape(1, c)
    as_r1 = att_src1.reshape(1, c)
    ad_r1 = att_dst1.reshape(1, c)
    b0r = b0.reshape(1, c)
    b1r = b1.reshape(1, c)
    g0r = gamma0.reshape(1, c)
    be0r = beta0.reshape(1, c)

    h0, as0, ad0 = _tc_prep(xp, W0, as_r0, ad_r0)
    num0, den0 = _sc_edge_pass(npad, gpw, src2d, dst2d,
                               as0.reshape(npad), ad0.reshape(npad), h0)
    h1, as1, ad1 = _tc_mid(n, num0, den0.reshape(NC, npad, 1), h0, as0, ad0,
                           b0r, g0r, be0r, W1, as_r1, ad_r1)
    num1, den1 = _sc_edge_pass(npad, gpw, src2d, dst2d,
                               as1.reshape(npad), ad1.reshape(npad), h1)
    return _tc_final(n, num1, den1.reshape(NC, npad, 1), h1, as1, ad1, b1r)
